# Initial kernel scaffold; baseline (speedup 1.0000x reference)
import jax
import jax.numpy as jnp
from jax.experimental import pallas as pl


def kernel(bilateral_grid, guidemap):
    raise NotImplementedError("write your pallas kernel here")


# speedup vs baseline: 1592.9492x; 1592.9492x over previous
import functools

import jax
import jax.numpy as jnp
from jax.experimental import pallas as pl
from jax.experimental.pallas import tpu as pltpu


def _slice_body(ga_ref, gb_ref, gc_ref, guide_ref, out_ref, *, scale, D, C, Wg, W):
    half = scale // 2
    wpos = jax.lax.broadcasted_iota(jnp.int32, (1, W), 1).astype(jnp.float32)
    gx = jnp.clip((wpos + 0.5) * (Wg / W) - 0.5, 0.0, Wg - 1.0)
    xg = jax.lax.broadcasted_iota(jnp.int32, (Wg, 1), 0).astype(jnp.float32)
    Bx = jnp.maximum(0.0, 1.0 - jnp.abs(gx - xg))

    ga = ga_ref[0, 0].reshape(C * D, Wg)
    gb = gb_ref[0, 0].reshape(C * D, Wg)
    gc = gc_ref[0, 0].reshape(C * D, Wg)
    A = jnp.dot(ga, Bx, preferred_element_type=jnp.float32)
    B = jnp.dot(gb, Bx, preferred_element_type=jnp.float32)
    Cc = jnp.dot(gc, Bx, preferred_element_type=jnp.float32)

    jrow = jax.lax.broadcasted_iota(jnp.int32, (half, 1), 0).astype(jnp.float32)
    for h in range(2):
        rows = guide_ref[0, 0, h * half:(h + 1) * half, :]
        gz = jnp.clip(rows * D - 0.5, 0.0, D - 1.0)
        if h == 0:
            base, diff = A, B - A
            wy = (jrow + 0.5) / scale + 0.5
        else:
            base, diff = B, Cc - B
            wy = (jrow + 0.5) / scale
        base = base.reshape(C, D, W)
        diff = diff.reshape(C, D, W)
        u = [jnp.maximum(0.0, 1.0 - jnp.abs(gz - d)) for d in range(D)]
        v = [u[d] * wy for d in range(D)]
        for c in range(C):
            acc = u[0] * base[c, 0][None, :] + v[0] * diff[c, 0][None, :]
            for d in range(1, D):
                acc = acc + u[d] * base[c, d][None, :]
                acc = acc + v[d] * diff[c, d][None, :]
            out_ref[0, c, h * half:(h + 1) * half, :] = acc


def kernel(bilateral_grid, guidemap):
    Bn, C, D, Hg, Wg = bilateral_grid.shape
    H, W = guidemap.shape[2], guidemap.shape[3]
    scale = H // Hg
    gridT = jnp.transpose(bilateral_grid, (0, 3, 1, 2, 4))

    body = functools.partial(_slice_body, scale=scale, D=D, C=C, Wg=Wg, W=W)
    grid = (Bn, Hg)

    def gmap(off):
        def imap(b, k):
            y = jnp.clip(k + off, 0, Hg - 1)
            return (b, y, 0, 0, 0)
        return imap

    out = pl.pallas_call(
        body,
        grid=grid,
        in_specs=[
            pl.BlockSpec((1, 1, C, D, Wg), gmap(-1)),
            pl.BlockSpec((1, 1, C, D, Wg), gmap(0)),
            pl.BlockSpec((1, 1, C, D, Wg), gmap(1)),
            pl.BlockSpec((1, 1, scale, W), lambda b, k: (b, 0, k, 0)),
        ],
        out_specs=pl.BlockSpec((1, C, scale, W), lambda b, k: (b, 0, k, 0)),
        out_shape=jax.ShapeDtypeStruct((Bn, C, H, W), jnp.float32),
        compiler_params=pltpu.CompilerParams(
            dimension_semantics=("parallel", "arbitrary"),
        ),
    )(gridT, gridT, gridT, guidemap)
    return out
